# Initial kernel scaffold; baseline (speedup 1.0000x reference)
import jax
import jax.numpy as jnp
from jax.experimental import pallas as pl


def kernel(x, edge_index, batch, W1, b1, W2, b2, Wout, bout):
    raise NotImplementedError("write your pallas kernel here")


# speedup vs baseline: 4.5687x; 4.5687x over previous
import functools

import jax
import jax.numpy as jnp
from jax import lax
from jax.experimental import pallas as pl
from jax.experimental.pallas import tpu as pltpu
from jax.experimental.pallas import tpu_sc as plsc

N = 10000
E = 320000
G = 64
DIN = 128
DH = 256
DOUT = 64
HALF = DH // 2

NC = 2
NS = 16
CHUNK = 80
NPAD = 10240
ROWS_PER_TILE = NPAD // NS

PASS_ROWS = NPAD // 2
ACC_ROWS = PASS_ROWS + 128
ZCHUNK = ACC_ROWS // NS
WCHUNK = PASS_ROWS // NS

_f32 = jnp.float32
_i32 = jnp.int32


def _sc_degree_body(dst_hbm, out_hbm, dbuf, ones_buf, wbuf, acc):
    cid = lax.axis_index("c")
    sid = lax.axis_index("s")

    def fill_ones(i, carry):
        for k in range(HALF // 16):
            ones_buf[i, pl.ds(k * 16, 16)] = jnp.full((16,), 1.0, _f32)
        return carry
    lax.fori_loop(0, CHUNK, fill_ones, 0)

    ept = E // (NC * NS)
    tile_base = (cid * NS + sid) * ept
    trash = jnp.arange(16, dtype=_i32) * 8 + PASS_ROWS

    for p in range(2):
        def fill_zero(i, carry):
            for k in range(HALF // 16):
                wbuf[i, pl.ds(k * 16, 16)] = jnp.zeros((16,), _f32)
            return carry
        lax.fori_loop(0, ZCHUNK, fill_zero, 0)
        pltpu.sync_copy(wbuf, acc.at[pl.ds(sid * ZCHUNK, ZCHUNK)])
        plsc.subcore_barrier()

        def body(j, carry):
            base = tile_base + j * CHUNK
            pltpu.sync_copy(dst_hbm.at[pl.ds(base, CHUNK)], dbuf)
            for k in range(CHUNK // 16):
                sl = pl.ds(k * 16, 16)
                d = dbuf[sl] - p * PASS_ROWS
                oob = (d < 0) | (d >= PASS_ROWS)
                dbuf[sl] = jnp.where(oob, trash, d)
            pltpu.sync_copy(ones_buf, acc.at[dbuf], add=True)
            return carry
        lax.fori_loop(0, ept // CHUNK, body, 0)
        plsc.subcore_barrier()

        row0 = sid * WCHUNK
        pltpu.sync_copy(acc.at[pl.ds(row0, WCHUNK)],
                        wbuf.at[pl.ds(0, WCHUNK)])
        pltpu.sync_copy(wbuf.at[pl.ds(0, WCHUNK)],
                        out_hbm.at[pl.ds(cid * NPAD + p * PASS_ROWS + row0,
                                         WCHUNK)])
        plsc.subcore_barrier()


def _sc_aggregate_body(src_hbm, dst_hbm, z_hbm, out_hbm,
                       sbuf, dbuf, rows, wbuf, acc, gsem):
    cid = lax.axis_index("c")
    sid = lax.axis_index("s")

    ept = E // NS
    tile_base = sid * ept
    plane_off = cid * N
    trash = jnp.arange(16, dtype=_i32) * 8 + PASS_ROWS

    for p in range(2):
        def fill_zero(i, carry):
            for k in range(HALF // 16):
                wbuf[i, pl.ds(k * 16, 16)] = jnp.zeros((16,), _f32)
            return carry
        lax.fori_loop(0, ZCHUNK, fill_zero, 0)
        pltpu.sync_copy(wbuf, acc.at[pl.ds(sid * ZCHUNK, ZCHUNK)])
        plsc.subcore_barrier()

        def body(j, carry):
            base = tile_base + j * CHUNK
            pltpu.sync_copy(src_hbm.at[pl.ds(base, CHUNK)], sbuf)
            pltpu.sync_copy(dst_hbm.at[pl.ds(base, CHUNK)], dbuf)
            for k in range(CHUNK // 16):
                sl = pl.ds(k * 16, 16)
                sbuf[sl] = sbuf[sl] + plane_off
                d = dbuf[sl] - p * PASS_ROWS
                oob = (d < 0) | (d >= PASS_ROWS)
                dbuf[sl] = jnp.where(oob, trash, d)
            pltpu.async_copy(z_hbm.at[sbuf], rows, gsem).wait()
            pltpu.sync_copy(rows, acc.at[dbuf], add=True)
            return carry
        lax.fori_loop(0, ept // CHUNK, body, 0)
        plsc.subcore_barrier()

        row0 = sid * WCHUNK
        pltpu.sync_copy(acc.at[pl.ds(row0, WCHUNK)],
                        wbuf.at[pl.ds(0, WCHUNK)])
        pltpu.sync_copy(wbuf.at[pl.ds(0, WCHUNK)],
                        out_hbm.at[pl.ds(cid * NPAD + p * PASS_ROWS + row0,
                                         WCHUNK)])
        plsc.subcore_barrier()


@functools.lru_cache(maxsize=1)
def _sc_kernels():
    mesh = plsc.VectorSubcoreMesh(core_axis_name="c", subcore_axis_name="s",
                                  num_cores=NC, num_subcores=NS)
    sc_degree = pl.kernel(
        _sc_degree_body,
        out_type=jax.ShapeDtypeStruct((NC * NPAD, HALF), _f32),
        mesh=mesh,
        scratch_types=[
            pltpu.VMEM((CHUNK,), _i32),
            pltpu.VMEM((CHUNK, HALF), _f32),
            pltpu.VMEM((ZCHUNK, HALF), _f32),
            pltpu.VMEM_SHARED((ACC_ROWS, HALF), _f32),
        ],
        name="sc_degree",
    )
    sc_aggregate = pl.kernel(
        _sc_aggregate_body,
        out_type=jax.ShapeDtypeStruct((NC * NPAD, HALF), _f32),
        mesh=mesh,
        scratch_types=[
            pltpu.VMEM((CHUNK,), _i32),
            pltpu.VMEM((CHUNK,), _i32),
            pltpu.VMEM((CHUNK, HALF), _f32),
            pltpu.VMEM((ZCHUNK, HALF), _f32),
            pltpu.VMEM_SHARED((ACC_ROWS, HALF), _f32),
            pltpu.SemaphoreType.DMA,
        ],
        name="sc_aggregate",
    )
    return sc_degree, sc_aggregate


_RB = 1000
_NB = N // _RB


def _tc_a_body(deg_ref, x_ref, w1_ref, z_ref, s_ref):
    d = deg_ref[0] + deg_ref[1] + 1.0
    s = lax.rsqrt(d)
    s_ref[...] = s
    s_col = s[:, :1]
    y = jnp.dot(x_ref[...], w1_ref[...], preferred_element_type=_f32)
    z = s_col * y
    z_ref[0] = z[:, :HALF]
    z_ref[1] = z[:, HALF:]


def _tc_a(deg3, x, W1):
    return pl.pallas_call(
        _tc_a_body,
        grid=(_NB,),
        in_specs=[
            pl.BlockSpec((2, _RB, HALF), lambda i: (0, i, 0)),
            pl.BlockSpec((_RB, DIN), lambda i: (i, 0)),
            pl.BlockSpec((DIN, DH), lambda i: (0, 0)),
        ],
        out_specs=[
            pl.BlockSpec((2, _RB, HALF), lambda i: (0, i, 0)),
            pl.BlockSpec((_RB, HALF), lambda i: (i, 0)),
        ],
        out_shape=[
            jax.ShapeDtypeStruct((2, N, HALF), _f32),
            jax.ShapeDtypeStruct((N, HALF), _f32),
        ],
    )(deg3, x, W1)


def _tc_b_body(agg_ref, z_ref, s_ref, b_ref, w_ref, zout_ref):
    a = jnp.concatenate([agg_ref[0], agg_ref[1]], axis=1)
    z = jnp.concatenate([z_ref[0], z_ref[1]], axis=1)
    s_col = s_ref[...][:, :1]
    h = jax.nn.relu(s_col * (a + z) + b_ref[...])
    y = jnp.dot(h, w_ref[...], preferred_element_type=_f32)
    zo = s_col * y
    zout_ref[0] = zo[:, :HALF]
    zout_ref[1] = zo[:, HALF:]


def _tc_b(agg, z, s, b, W):
    return pl.pallas_call(
        _tc_b_body,
        grid=(_NB,),
        in_specs=[
            pl.BlockSpec((2, _RB, HALF), lambda i: (0, i, 0)),
            pl.BlockSpec((2, _RB, HALF), lambda i: (0, i, 0)),
            pl.BlockSpec((_RB, HALF), lambda i: (i, 0)),
            pl.BlockSpec((1, DH), lambda i: (0, 0)),
            pl.BlockSpec((DH, DH), lambda i: (0, 0)),
        ],
        out_specs=pl.BlockSpec((2, _RB, HALF), lambda i: (0, i, 0)),
        out_shape=jax.ShapeDtypeStruct((2, N, HALF), _f32),
    )(agg, z, s, b, W)


def _tc_c_body(agg_ref, z_ref, s_ref, b_ref, batch_ref, wout_ref, bout_ref,
               out_ref, pooled, cnt):
    i = pl.program_id(0)

    @pl.when(i == 0)
    def _init():
        pooled[...] = jnp.zeros((G, DH), _f32)
        cnt[...] = jnp.zeros((G, 128), _f32)

    a = jnp.concatenate([agg_ref[0], agg_ref[1]], axis=1)
    z = jnp.concatenate([z_ref[0], z_ref[1]], axis=1)
    s_col = s_ref[...][:, :1]
    h = jax.nn.relu(s_col * (a + z) + b_ref[...])

    bblk = batch_ref[0]
    gid = lax.broadcasted_iota(_i32, (G, _RB), 0)
    pt = (gid == bblk).astype(_f32)
    pooled[...] += jnp.dot(pt, h, preferred_element_type=_f32)
    rowsum = jnp.sum(pt, axis=1, keepdims=True)
    cnt[...] += jnp.broadcast_to(rowsum, (G, 128))

    @pl.when(i == _NB - 1)
    def _finish():
        denom = jnp.clip(cnt[...][:, :1], 1.0, None)
        avg = pooled[...] / denom
        logits = jnp.dot(avg, wout_ref[...],
                         preferred_element_type=_f32) + bout_ref[...]
        out_ref[...] = jax.nn.sigmoid(logits)


def _tc_c(agg, z, s, b, batch3, Wout, bout):
    return pl.pallas_call(
        _tc_c_body,
        grid=(_NB,),
        in_specs=[
            pl.BlockSpec((2, _RB, HALF), lambda i: (0, i, 0)),
            pl.BlockSpec((2, _RB, HALF), lambda i: (0, i, 0)),
            pl.BlockSpec((_RB, HALF), lambda i: (i, 0)),
            pl.BlockSpec((1, DH), lambda i: (0, 0)),
            pl.BlockSpec((1, 1, _RB), lambda i: (i, 0, 0)),
            pl.BlockSpec((DH, DOUT), lambda i: (0, 0)),
            pl.BlockSpec((1, DOUT), lambda i: (0, 0)),
        ],
        out_specs=pl.BlockSpec((G, DOUT), lambda i: (0, 0)),
        out_shape=jax.ShapeDtypeStruct((G, DOUT), _f32),
        scratch_shapes=[
            pltpu.VMEM((G, DH), _f32),
            pltpu.VMEM((G, 128), _f32),
        ],
    )(agg, z, s, b, batch3, Wout, bout)


def kernel(x, edge_index, batch, W1, b1, W2, b2, Wout, bout):
    src = edge_index[0].astype(_i32)
    dst = edge_index[1].astype(_i32)
    batch3 = batch.astype(_i32).reshape(_NB, 1, _RB)

    sc_degree, sc_aggregate = _sc_kernels()
    deg2 = sc_degree(dst)
    deg3 = deg2.reshape(2, NPAD, HALF)

    z1, s = _tc_a(deg3, x.astype(_f32), W1)
    agg1 = sc_aggregate(src, dst, z1.reshape(NC * N, HALF))
    z2 = _tc_b(agg1.reshape(2, NPAD, HALF), z1, s, b1.reshape(1, DH), W2)
    agg2 = sc_aggregate(src, dst, z2.reshape(NC * N, HALF))
    return _tc_c(agg2.reshape(2, NPAD, HALF), z2, s, b2.reshape(1, DH),
                 batch3, Wout, bout.reshape(1, DOUT))


# speedup vs baseline: 8.6021x; 1.8828x over previous
import functools

import jax
import jax.numpy as jnp
from jax import lax
from jax.experimental import pallas as pl
from jax.experimental.pallas import tpu as pltpu
from jax.experimental.pallas import tpu_sc as plsc

N = 10000
E = 320000
G = 64
DIN = 128
DH = 256
DOUT = 64
HALF = DH // 2

NC = 2
NS = 16
CHUNK = 80
NPAD = 10240
ROWS_PER_TILE = NPAD // NS

PASS_ROWS = NPAD // 2
ACC_ROWS = PASS_ROWS + 128
ZCHUNK = ACC_ROWS // NS
WCHUNK = PASS_ROWS // NS
EPT = E // NS
SUP = 10

_f32 = jnp.float32
_i32 = jnp.int32


def _sc_degree_body(dst_hbm, out_hbm, dbuf, ones_buf, wbuf, acc):
    cid = lax.axis_index("c")
    sid = lax.axis_index("s")

    def fill_ones(i, carry):
        for k in range(HALF // 16):
            ones_buf[i, pl.ds(k * 16, 16)] = jnp.full((16,), 1.0, _f32)
        return carry
    lax.fori_loop(0, CHUNK, fill_ones, 0)

    ept = E // (NC * NS)
    tile_base = (cid * NS + sid) * ept
    trash = jnp.arange(16, dtype=_i32) * 8 + PASS_ROWS

    for p in range(2):
        def fill_zero(i, carry):
            for k in range(HALF // 16):
                wbuf[i, pl.ds(k * 16, 16)] = jnp.zeros((16,), _f32)
            return carry
        lax.fori_loop(0, ZCHUNK, fill_zero, 0)
        pltpu.sync_copy(wbuf, acc.at[pl.ds(sid * ZCHUNK, ZCHUNK)])
        plsc.subcore_barrier()

        def body(j, carry):
            base = tile_base + j * CHUNK
            pltpu.sync_copy(dst_hbm.at[pl.ds(base, CHUNK)], dbuf)
            for k in range(CHUNK // 16):
                sl = pl.ds(k * 16, 16)
                d = dbuf[sl] - p * PASS_ROWS
                oob = (d < 0) | (d >= PASS_ROWS)
                dbuf[sl] = jnp.where(oob, trash, d)
            pltpu.sync_copy(ones_buf, acc.at[dbuf], add=True)
            return carry
        lax.fori_loop(0, ept // CHUNK, body, 0)
        plsc.subcore_barrier()

        row0 = sid * WCHUNK
        pltpu.sync_copy(acc.at[pl.ds(row0, WCHUNK)],
                        wbuf.at[pl.ds(0, WCHUNK)])
        pltpu.sync_copy(wbuf.at[pl.ds(0, WCHUNK)],
                        out_hbm.at[pl.ds(cid * NPAD + p * PASS_ROWS + row0,
                                         WCHUNK)])
        plsc.subcore_barrier()


def _sc_aggregate_body(src_hbm, dst_hbm, z_hbm, out_hbm,
                       sstage, dstage, stab, dtab, rows0, rows1, wbuf, acc,
                       sem0, sem1):
    cid = lax.axis_index("c")
    sid = lax.axis_index("s")

    ept = E // NS
    tile_base = sid * ept
    plane_off = cid * N
    trash = jnp.arange(16, dtype=_i32) * 8 + PASS_ROWS
    rows = (rows0, rows1)
    sems = (sem0, sem1)

    def drain(rbuf, sem):
        pltpu.make_async_copy(z_hbm.at[stab.at[0]], rbuf, sem).wait()

    for p in range(2):
        def fill_zero(i, carry):
            for k in range(HALF // 16):
                wbuf[i, pl.ds(k * 16, 16)] = jnp.zeros((16,), _f32)
            return carry
        lax.fori_loop(0, CHUNK, fill_zero, 0)
        z0 = sid * ZCHUNK
        for off in (0, 80, 160, 240):
            pltpu.sync_copy(wbuf, acc.at[pl.ds(z0 + off, CHUNK)])
        pltpu.sync_copy(wbuf.at[pl.ds(0, ZCHUNK - 320)],
                        acc.at[pl.ds(z0 + 320, ZCHUNK - 320)])
        plsc.subcore_barrier()

        def sbody(sj, carry):
            base = tile_base + sj * (SUP * CHUNK)
            pltpu.sync_copy(src_hbm.at[pl.ds(base, SUP * CHUNK)], sstage)
            pltpu.sync_copy(dst_hbm.at[pl.ds(base, SUP * CHUNK)], dstage)
            for t in range(SUP):
                for i in range(CHUNK // 16):
                    sl = pl.ds(t * CHUNK + i * 16, 16)
                    stab[t, pl.ds(i * 16, 16)] = sstage[sl] + plane_off
                    d = dstage[sl] - p * PASS_ROWS
                    oob = (d < 0) | (d >= PASS_ROWS)
                    dtab[t, pl.ds(i * 16, 16)] = jnp.where(oob, trash, d)
            pltpu.async_copy(z_hbm.at[stab.at[0]], rows0, sem0)
            for t in range(SUP):
                if t + 1 < SUP:
                    pltpu.async_copy(z_hbm.at[stab.at[t + 1]],
                                     rows[(t + 1) % 2], sems[(t + 1) % 2])
                drain(rows[t % 2], sems[t % 2])
                pltpu.sync_copy(rows[t % 2], acc.at[dtab.at[t]], add=True)
            return carry
        lax.fori_loop(0, ept // (SUP * CHUNK), sbody, 0)
        plsc.subcore_barrier()

        row0 = sid * WCHUNK
        out0 = cid * NPAD + p * PASS_ROWS + row0
        for off in (0, 80, 160, 240):
            pltpu.sync_copy(acc.at[pl.ds(row0 + off, CHUNK)], wbuf)
            pltpu.sync_copy(wbuf, out_hbm.at[pl.ds(out0 + off, CHUNK)])
        plsc.subcore_barrier()


@functools.lru_cache(maxsize=1)
def _sc_kernels():
    mesh = plsc.VectorSubcoreMesh(core_axis_name="c", subcore_axis_name="s",
                                  num_cores=NC, num_subcores=NS)
    sc_degree = pl.kernel(
        _sc_degree_body,
        out_type=jax.ShapeDtypeStruct((NC * NPAD, HALF), _f32),
        mesh=mesh,
        scratch_types=[
            pltpu.VMEM((CHUNK,), _i32),
            pltpu.VMEM((CHUNK, HALF), _f32),
            pltpu.VMEM((ZCHUNK, HALF), _f32),
            pltpu.VMEM_SHARED((ACC_ROWS, HALF), _f32),
        ],
        name="sc_degree",
    )
    sc_aggregate = pl.kernel(
        _sc_aggregate_body,
        out_type=jax.ShapeDtypeStruct((NC * NPAD, HALF), _f32),
        mesh=mesh,
        scratch_types=[
            pltpu.VMEM((SUP * CHUNK,), _i32),
            pltpu.VMEM((SUP * CHUNK,), _i32),
            pltpu.VMEM((SUP, CHUNK), _i32),
            pltpu.VMEM((SUP, CHUNK), _i32),
            pltpu.VMEM((CHUNK, HALF), _f32),
            pltpu.VMEM((CHUNK, HALF), _f32),
            pltpu.VMEM((CHUNK, HALF), _f32),
            pltpu.VMEM_SHARED((ACC_ROWS, HALF), _f32),
            pltpu.SemaphoreType.DMA,
            pltpu.SemaphoreType.DMA,
        ],
        name="sc_aggregate",
    )
    return sc_degree, sc_aggregate


_RB = 1000
_NB = N // _RB


def _tc_a_body(deg_ref, x_ref, w1_ref, z_ref, s_ref):
    d = deg_ref[0] + deg_ref[1] + 1.0
    s = lax.rsqrt(d)
    s_ref[...] = s
    s_col = s[:, :1]
    y = jnp.dot(x_ref[...], w1_ref[...], preferred_element_type=_f32)
    z = s_col * y
    z_ref[0] = z[:, :HALF]
    z_ref[1] = z[:, HALF:]


def _tc_a(deg3, x, W1):
    return pl.pallas_call(
        _tc_a_body,
        grid=(_NB,),
        in_specs=[
            pl.BlockSpec((2, _RB, HALF), lambda i: (0, i, 0)),
            pl.BlockSpec((_RB, DIN), lambda i: (i, 0)),
            pl.BlockSpec((DIN, DH), lambda i: (0, 0)),
        ],
        out_specs=[
            pl.BlockSpec((2, _RB, HALF), lambda i: (0, i, 0)),
            pl.BlockSpec((_RB, HALF), lambda i: (i, 0)),
        ],
        out_shape=[
            jax.ShapeDtypeStruct((2, N, HALF), _f32),
            jax.ShapeDtypeStruct((N, HALF), _f32),
        ],
    )(deg3, x, W1)


def _tc_b_body(agg_ref, z_ref, s_ref, b_ref, w_ref, zout_ref):
    a = jnp.concatenate([agg_ref[0], agg_ref[1]], axis=1)
    z = jnp.concatenate([z_ref[0], z_ref[1]], axis=1)
    s_col = s_ref[...][:, :1]
    h = jax.nn.relu(s_col * (a + z) + b_ref[...])
    y = jnp.dot(h, w_ref[...], preferred_element_type=_f32)
    zo = s_col * y
    zout_ref[0] = zo[:, :HALF]
    zout_ref[1] = zo[:, HALF:]


def _tc_b(agg, z, s, b, W):
    return pl.pallas_call(
        _tc_b_body,
        grid=(_NB,),
        in_specs=[
            pl.BlockSpec((2, _RB, HALF), lambda i: (0, i, 0)),
            pl.BlockSpec((2, _RB, HALF), lambda i: (0, i, 0)),
            pl.BlockSpec((_RB, HALF), lambda i: (i, 0)),
            pl.BlockSpec((1, DH), lambda i: (0, 0)),
            pl.BlockSpec((DH, DH), lambda i: (0, 0)),
        ],
        out_specs=pl.BlockSpec((2, _RB, HALF), lambda i: (0, i, 0)),
        out_shape=jax.ShapeDtypeStruct((2, N, HALF), _f32),
    )(agg, z, s, b, W)


def _tc_c_body(agg_ref, z_ref, s_ref, b_ref, batch_ref, wout_ref, bout_ref,
               out_ref, pooled, cnt):
    i = pl.program_id(0)

    @pl.when(i == 0)
    def _init():
        pooled[...] = jnp.zeros((G, DH), _f32)
        cnt[...] = jnp.zeros((G, 128), _f32)

    a = jnp.concatenate([agg_ref[0], agg_ref[1]], axis=1)
    z = jnp.concatenate([z_ref[0], z_ref[1]], axis=1)
    s_col = s_ref[...][:, :1]
    h = jax.nn.relu(s_col * (a + z) + b_ref[...])

    bblk = batch_ref[0]
    gid = lax.broadcasted_iota(_i32, (G, _RB), 0)
    pt = (gid == bblk).astype(_f32)
    pooled[...] += jnp.dot(pt, h, preferred_element_type=_f32)
    rowsum = jnp.sum(pt, axis=1, keepdims=True)
    cnt[...] += jnp.broadcast_to(rowsum, (G, 128))

    @pl.when(i == _NB - 1)
    def _finish():
        denom = jnp.clip(cnt[...][:, :1], 1.0, None)
        avg = pooled[...] / denom
        logits = jnp.dot(avg, wout_ref[...],
                         preferred_element_type=_f32) + bout_ref[...]
        out_ref[...] = jax.nn.sigmoid(logits)


def _tc_c(agg, z, s, b, batch3, Wout, bout):
    return pl.pallas_call(
        _tc_c_body,
        grid=(_NB,),
        in_specs=[
            pl.BlockSpec((2, _RB, HALF), lambda i: (0, i, 0)),
            pl.BlockSpec((2, _RB, HALF), lambda i: (0, i, 0)),
            pl.BlockSpec((_RB, HALF), lambda i: (i, 0)),
            pl.BlockSpec((1, DH), lambda i: (0, 0)),
            pl.BlockSpec((1, 1, _RB), lambda i: (i, 0, 0)),
            pl.BlockSpec((DH, DOUT), lambda i: (0, 0)),
            pl.BlockSpec((1, DOUT), lambda i: (0, 0)),
        ],
        out_specs=pl.BlockSpec((G, DOUT), lambda i: (0, 0)),
        out_shape=jax.ShapeDtypeStruct((G, DOUT), _f32),
        scratch_shapes=[
            pltpu.VMEM((G, DH), _f32),
            pltpu.VMEM((G, 128), _f32),
        ],
    )(agg, z, s, b, batch3, Wout, bout)


def kernel(x, edge_index, batch, W1, b1, W2, b2, Wout, bout):
    src = edge_index[0].astype(_i32)
    dst = edge_index[1].astype(_i32)
    batch3 = batch.astype(_i32).reshape(_NB, 1, _RB)

    sc_degree, sc_aggregate = _sc_kernels()
    deg2 = sc_degree(dst)
    deg3 = deg2.reshape(2, NPAD, HALF)

    z1, s = _tc_a(deg3, x.astype(_f32), W1)
    agg1 = sc_aggregate(src, dst, z1.reshape(NC * N, HALF))
    z2 = _tc_b(agg1.reshape(2, NPAD, HALF), z1, s, b1.reshape(1, DH), W2)
    agg2 = sc_aggregate(src, dst, z2.reshape(NC * N, HALF))
    return _tc_c(agg2.reshape(2, NPAD, HALF), z2, s, b2.reshape(1, DH),
                 batch3, Wout, bout.reshape(1, DOUT))


# speedup vs baseline: 8.8260x; 1.0260x over previous
import functools

import jax
import jax.numpy as jnp
from jax import lax
from jax.experimental import pallas as pl
from jax.experimental.pallas import tpu as pltpu
from jax.experimental.pallas import tpu_sc as plsc

N = 10000
E = 320000
G = 64
DIN = 128
DH = 256
DOUT = 64
HALF = DH // 2

NC = 2
NS = 16
CHUNK = 80
NPAD = 10240
ROWS_PER_TILE = NPAD // NS

PASS_ROWS = NPAD // 2
ACC_ROWS = PASS_ROWS + 128
ZCHUNK = ACC_ROWS // NS
WCHUNK = PASS_ROWS // NS
EPT = E // NS
SUP = 10
DSUP = 25

_f32 = jnp.float32
_i32 = jnp.int32


def _sc_degree_body(dst_hbm, out_hbm, dstage, dtab, ones_buf, wbuf, acc,
                    sem):
    cid = lax.axis_index("c")
    sid = lax.axis_index("s")

    def fill_ones(i, carry):
        for k in range(HALF // 16):
            ones_buf[i, pl.ds(k * 16, 16)] = jnp.full((16,), 1.0, _f32)
        return carry
    lax.fori_loop(0, CHUNK, fill_ones, 0)

    ept = E // (NC * NS)
    tile_base = (cid * NS + sid) * ept
    trash = jnp.arange(16, dtype=_i32) * 8 + PASS_ROWS

    for p in range(2):
        def fill_zero(i, carry):
            for k in range(HALF // 16):
                wbuf[i, pl.ds(k * 16, 16)] = jnp.zeros((16,), _f32)
            return carry
        lax.fori_loop(0, CHUNK, fill_zero, 0)
        z0 = sid * ZCHUNK
        for off in (0, 80, 160, 240):
            pltpu.sync_copy(wbuf, acc.at[pl.ds(z0 + off, CHUNK)])
        pltpu.sync_copy(wbuf.at[pl.ds(0, ZCHUNK - 320)],
                        acc.at[pl.ds(z0 + 320, ZCHUNK - 320)])
        plsc.subcore_barrier()

        def sbody(sj, carry):
            base = tile_base + sj * (DSUP * CHUNK)
            pltpu.sync_copy(dst_hbm.at[pl.ds(base, DSUP * CHUNK)], dstage)
            for t in range(DSUP):
                for i in range(CHUNK // 16):
                    d = dstage[pl.ds(t * CHUNK + i * 16, 16)] - p * PASS_ROWS
                    oob = (d < 0) | (d >= PASS_ROWS)
                    dtab[t, pl.ds(i * 16, 16)] = jnp.where(oob, trash, d)
            for t in range(DSUP):
                pltpu.async_copy(ones_buf, acc.at[dtab.at[t]], sem,
                                 add=True)
            for t in range(DSUP):
                pltpu.make_async_copy(out_hbm.at[pl.ds(0, CHUNK)],
                                      ones_buf, sem).wait()
            return carry
        lax.fori_loop(0, ept // (DSUP * CHUNK), sbody, 0)
        plsc.subcore_barrier()

        row0 = sid * WCHUNK
        out0 = cid * NPAD + p * PASS_ROWS + row0
        for off in (0, 80, 160, 240):
            pltpu.sync_copy(acc.at[pl.ds(row0 + off, CHUNK)], wbuf)
            pltpu.sync_copy(wbuf, out_hbm.at[pl.ds(out0 + off, CHUNK)])
        plsc.subcore_barrier()


def _sc_aggregate_body(src_hbm, dst_hbm, z_hbm, out_hbm,
                       sstage, dstage, stab, dtab, rows0, rows1, rows2,
                       rows3, wbuf, acc, sem0, sem1, sem2, sem3):
    cid = lax.axis_index("c")
    sid = lax.axis_index("s")

    ept = E // NS
    tile_base = sid * ept
    plane_off = cid * N
    trash = jnp.arange(16, dtype=_i32) * 8 + PASS_ROWS
    rows = (rows0, rows1, rows2, rows3)
    sems = (sem0, sem1, sem2, sem3)

    def drain(rbuf, sem):
        pltpu.make_async_copy(z_hbm.at[stab.at[0]], rbuf, sem).wait()

    for p in range(2):
        def fill_zero(i, carry):
            for k in range(HALF // 16):
                wbuf[i, pl.ds(k * 16, 16)] = jnp.zeros((16,), _f32)
            return carry
        lax.fori_loop(0, CHUNK, fill_zero, 0)
        z0 = sid * ZCHUNK
        for off in (0, 80, 160, 240):
            pltpu.sync_copy(wbuf, acc.at[pl.ds(z0 + off, CHUNK)])
        pltpu.sync_copy(wbuf.at[pl.ds(0, ZCHUNK - 320)],
                        acc.at[pl.ds(z0 + 320, ZCHUNK - 320)])
        plsc.subcore_barrier()

        def sbody(sj, carry):
            base = tile_base + sj * (SUP * CHUNK)
            pltpu.sync_copy(src_hbm.at[pl.ds(base, SUP * CHUNK)], sstage)
            pltpu.sync_copy(dst_hbm.at[pl.ds(base, SUP * CHUNK)], dstage)
            for t in range(SUP):
                for i in range(CHUNK // 16):
                    sl = pl.ds(t * CHUNK + i * 16, 16)
                    stab[t, pl.ds(i * 16, 16)] = sstage[sl] + plane_off
                    d = dstage[sl] - p * PASS_ROWS
                    oob = (d < 0) | (d >= PASS_ROWS)
                    dtab[t, pl.ds(i * 16, 16)] = jnp.where(oob, trash, d)
            for t in range(3):
                pltpu.async_copy(z_hbm.at[stab.at[t]], rows[t], sems[t])
            for t in range(SUP):
                b = t % 4
                drain(rows[b], sems[b])
                pltpu.async_copy(rows[b], acc.at[dtab.at[t]],
                                 sems[b], add=True)
                nt = t + 3
                if nt < SUP:
                    nb = nt % 4
                    if t >= 1:
                        drain(rows[nb], sems[nb])
                    pltpu.async_copy(z_hbm.at[stab.at[nt]], rows[nb],
                                     sems[nb])
            for t in range(SUP - 4, SUP):
                drain(rows[t % 4], sems[t % 4])
            return carry
        lax.fori_loop(0, ept // (SUP * CHUNK), sbody, 0)
        plsc.subcore_barrier()

        row0 = sid * WCHUNK
        out0 = cid * NPAD + p * PASS_ROWS + row0
        for off in (0, 80, 160, 240):
            pltpu.sync_copy(acc.at[pl.ds(row0 + off, CHUNK)], wbuf)
            pltpu.sync_copy(wbuf, out_hbm.at[pl.ds(out0 + off, CHUNK)])
        plsc.subcore_barrier()


@functools.lru_cache(maxsize=1)
def _sc_kernels():
    mesh = plsc.VectorSubcoreMesh(core_axis_name="c", subcore_axis_name="s",
                                  num_cores=NC, num_subcores=NS)
    sc_degree = pl.kernel(
        _sc_degree_body,
        out_type=jax.ShapeDtypeStruct((NC * NPAD, HALF), _f32),
        mesh=mesh,
        scratch_types=[
            pltpu.VMEM((DSUP * CHUNK,), _i32),
            pltpu.VMEM((DSUP, CHUNK), _i32),
            pltpu.VMEM((CHUNK, HALF), _f32),
            pltpu.VMEM((CHUNK, HALF), _f32),
            pltpu.VMEM_SHARED((ACC_ROWS, HALF), _f32),
            pltpu.SemaphoreType.DMA,
        ],
        name="sc_degree",
    )
    sc_aggregate = pl.kernel(
        _sc_aggregate_body,
        out_type=jax.ShapeDtypeStruct((NC * NPAD, HALF), _f32),
        mesh=mesh,
        scratch_types=[
            pltpu.VMEM((SUP * CHUNK,), _i32),
            pltpu.VMEM((SUP * CHUNK,), _i32),
            pltpu.VMEM((SUP, CHUNK), _i32),
            pltpu.VMEM((SUP, CHUNK), _i32),
            pltpu.VMEM((CHUNK, HALF), _f32),
            pltpu.VMEM((CHUNK, HALF), _f32),
            pltpu.VMEM((CHUNK, HALF), _f32),
            pltpu.VMEM((CHUNK, HALF), _f32),
            pltpu.VMEM((CHUNK, HALF), _f32),
            pltpu.VMEM_SHARED((ACC_ROWS, HALF), _f32),
            pltpu.SemaphoreType.DMA,
            pltpu.SemaphoreType.DMA,
            pltpu.SemaphoreType.DMA,
            pltpu.SemaphoreType.DMA,
        ],
        name="sc_aggregate",
    )
    return sc_degree, sc_aggregate


_RB = 1000
_NB = N // _RB


def _tc_a_body(deg_ref, x_ref, w1_ref, z_ref, s_ref):
    d = deg_ref[0] + deg_ref[1] + 1.0
    s = lax.rsqrt(d)
    s_ref[...] = s
    s_col = s[:, :1]
    y = jnp.dot(x_ref[...], w1_ref[...], preferred_element_type=_f32)
    z = s_col * y
    z_ref[0] = z[:, :HALF]
    z_ref[1] = z[:, HALF:]


def _tc_a(deg3, x, W1):
    return pl.pallas_call(
        _tc_a_body,
        grid=(_NB,),
        in_specs=[
            pl.BlockSpec((2, _RB, HALF), lambda i: (0, i, 0)),
            pl.BlockSpec((_RB, DIN), lambda i: (i, 0)),
            pl.BlockSpec((DIN, DH), lambda i: (0, 0)),
        ],
        out_specs=[
            pl.BlockSpec((2, _RB, HALF), lambda i: (0, i, 0)),
            pl.BlockSpec((_RB, HALF), lambda i: (i, 0)),
        ],
        out_shape=[
            jax.ShapeDtypeStruct((2, N, HALF), _f32),
            jax.ShapeDtypeStruct((N, HALF), _f32),
        ],
    )(deg3, x, W1)


def _tc_b_body(agg_ref, z_ref, s_ref, b_ref, w_ref, zout_ref):
    a = jnp.concatenate([agg_ref[0], agg_ref[1]], axis=1)
    z = jnp.concatenate([z_ref[0], z_ref[1]], axis=1)
    s_col = s_ref[...][:, :1]
    h = jax.nn.relu(s_col * (a + z) + b_ref[...])
    y = jnp.dot(h, w_ref[...], preferred_element_type=_f32)
    zo = s_col * y
    zout_ref[0] = zo[:, :HALF]
    zout_ref[1] = zo[:, HALF:]


def _tc_b(agg, z, s, b, W):
    return pl.pallas_call(
        _tc_b_body,
        grid=(_NB,),
        in_specs=[
            pl.BlockSpec((2, _RB, HALF), lambda i: (0, i, 0)),
            pl.BlockSpec((2, _RB, HALF), lambda i: (0, i, 0)),
            pl.BlockSpec((_RB, HALF), lambda i: (i, 0)),
            pl.BlockSpec((1, DH), lambda i: (0, 0)),
            pl.BlockSpec((DH, DH), lambda i: (0, 0)),
        ],
        out_specs=pl.BlockSpec((2, _RB, HALF), lambda i: (0, i, 0)),
        out_shape=jax.ShapeDtypeStruct((2, N, HALF), _f32),
    )(agg, z, s, b, W)


def _tc_c_body(agg_ref, z_ref, s_ref, b_ref, batch_ref, wout_ref, bout_ref,
               out_ref, pooled, cnt):
    i = pl.program_id(0)

    @pl.when(i == 0)
    def _init():
        pooled[...] = jnp.zeros((G, DH), _f32)
        cnt[...] = jnp.zeros((G, 128), _f32)

    a = jnp.concatenate([agg_ref[0], agg_ref[1]], axis=1)
    z = jnp.concatenate([z_ref[0], z_ref[1]], axis=1)
    s_col = s_ref[...][:, :1]
    h = jax.nn.relu(s_col * (a + z) + b_ref[...])

    bblk = batch_ref[0]
    gid = lax.broadcasted_iota(_i32, (G, _RB), 0)
    pt = (gid == bblk).astype(_f32)
    pooled[...] += jnp.dot(pt, h, preferred_element_type=_f32)
    rowsum = jnp.sum(pt, axis=1, keepdims=True)
    cnt[...] += jnp.broadcast_to(rowsum, (G, 128))

    @pl.when(i == _NB - 1)
    def _finish():
        denom = jnp.clip(cnt[...][:, :1], 1.0, None)
        avg = pooled[...] / denom
        logits = jnp.dot(avg, wout_ref[...],
                         preferred_element_type=_f32) + bout_ref[...]
        out_ref[...] = jax.nn.sigmoid(logits)


def _tc_c(agg, z, s, b, batch3, Wout, bout):
    return pl.pallas_call(
        _tc_c_body,
        grid=(_NB,),
        in_specs=[
            pl.BlockSpec((2, _RB, HALF), lambda i: (0, i, 0)),
            pl.BlockSpec((2, _RB, HALF), lambda i: (0, i, 0)),
            pl.BlockSpec((_RB, HALF), lambda i: (i, 0)),
            pl.BlockSpec((1, DH), lambda i: (0, 0)),
            pl.BlockSpec((1, 1, _RB), lambda i: (i, 0, 0)),
            pl.BlockSpec((DH, DOUT), lambda i: (0, 0)),
            pl.BlockSpec((1, DOUT), lambda i: (0, 0)),
        ],
        out_specs=pl.BlockSpec((G, DOUT), lambda i: (0, 0)),
        out_shape=jax.ShapeDtypeStruct((G, DOUT), _f32),
        scratch_shapes=[
            pltpu.VMEM((G, DH), _f32),
            pltpu.VMEM((G, 128), _f32),
        ],
    )(agg, z, s, b, batch3, Wout, bout)


def kernel(x, edge_index, batch, W1, b1, W2, b2, Wout, bout):
    src = edge_index[0].astype(_i32)
    dst = edge_index[1].astype(_i32)
    batch3 = batch.astype(_i32).reshape(_NB, 1, _RB)

    sc_degree, sc_aggregate = _sc_kernels()
    deg2 = sc_degree(dst)
    deg3 = deg2.reshape(2, NPAD, HALF)

    z1, s = _tc_a(deg3, x.astype(_f32), W1)
    agg1 = sc_aggregate(src, dst, z1.reshape(NC * N, HALF))
    z2 = _tc_b(agg1.reshape(2, NPAD, HALF), z1, s, b1.reshape(1, DH), W2)
    agg2 = sc_aggregate(src, dst, z2.reshape(NC * N, HALF))
    return _tc_c(agg2.reshape(2, NPAD, HALF), z2, s, b2.reshape(1, DH),
                 batch3, Wout, bout.reshape(1, DOUT))


# speedup vs baseline: 9.4739x; 1.0734x over previous
import functools

import jax
import jax.numpy as jnp
from jax import lax
from jax.experimental import pallas as pl
from jax.experimental.pallas import tpu as pltpu
from jax.experimental.pallas import tpu_sc as plsc

N = 10000
E = 320000
G = 64
DIN = 128
DH = 256
DOUT = 64
HALF = DH // 2

NC = 2
NS = 16
CHUNK = 80
NPAD = 10240
ROWS_PER_TILE = NPAD // NS

PASS_ROWS = NPAD // 2
ACC_ROWS = PASS_ROWS + 128
ZCHUNK = ACC_ROWS // NS
WCHUNK = PASS_ROWS // NS
EPT = E // NS
SUP = 25
DSUP = 25

_f32 = jnp.float32
_i32 = jnp.int32


def _sc_degree_body(dst_hbm, out_hbm, dstage, dtab, ones_buf, wbuf, acc,
                    sem):
    cid = lax.axis_index("c")
    sid = lax.axis_index("s")

    def fill_ones(i, carry):
        for k in range(HALF // 16):
            ones_buf[i, pl.ds(k * 16, 16)] = jnp.full((16,), 1.0, _f32)
        return carry
    lax.fori_loop(0, CHUNK, fill_ones, 0)

    ept = E // (NC * NS)
    tile_base = (cid * NS + sid) * ept
    trash = jnp.arange(16, dtype=_i32) * 8 + PASS_ROWS

    for p in range(2):
        def fill_zero(i, carry):
            for k in range(HALF // 16):
                wbuf[i, pl.ds(k * 16, 16)] = jnp.zeros((16,), _f32)
            return carry
        lax.fori_loop(0, CHUNK, fill_zero, 0)
        z0 = sid * ZCHUNK
        for off in (0, 80, 160, 240):
            pltpu.sync_copy(wbuf, acc.at[pl.ds(z0 + off, CHUNK)])
        pltpu.sync_copy(wbuf.at[pl.ds(0, ZCHUNK - 320)],
                        acc.at[pl.ds(z0 + 320, ZCHUNK - 320)])
        plsc.subcore_barrier()

        def sbody(sj, carry):
            base = tile_base + sj * (DSUP * CHUNK)
            pltpu.sync_copy(dst_hbm.at[pl.ds(base, DSUP * CHUNK)], dstage)
            for t in range(DSUP):
                for i in range(CHUNK // 16):
                    d = dstage[pl.ds(t * CHUNK + i * 16, 16)] - p * PASS_ROWS
                    oob = (d < 0) | (d >= PASS_ROWS)
                    dtab[t, pl.ds(i * 16, 16)] = jnp.where(oob, trash, d)
            for t in range(DSUP):
                pltpu.async_copy(ones_buf, acc.at[dtab.at[t]], sem,
                                 add=True)
            for t in range(DSUP):
                pltpu.make_async_copy(out_hbm.at[pl.ds(0, CHUNK)],
                                      ones_buf, sem).wait()
            return carry
        lax.fori_loop(0, ept // (DSUP * CHUNK), sbody, 0)
        plsc.subcore_barrier()

        row0 = sid * WCHUNK
        out0 = cid * NPAD + p * PASS_ROWS + row0
        for off in (0, 80, 160, 240):
            pltpu.sync_copy(acc.at[pl.ds(row0 + off, CHUNK)], wbuf)
            pltpu.sync_copy(wbuf, out_hbm.at[pl.ds(out0 + off, CHUNK)])
        plsc.subcore_barrier()


def _sc_aggregate_body(src_hbm, dst_hbm, z_hbm, out_hbm,
                       sstage, dstage, stab, dtab, rows0, rows1, rows2,
                       rows3, wbuf, acc, sem0, sem1, sem2, sem3):
    cid = lax.axis_index("c")
    sid = lax.axis_index("s")

    ept = E // NS
    tile_base = sid * ept
    plane_off = cid * N
    trash = jnp.arange(16, dtype=_i32) * 8 + PASS_ROWS
    rows = (rows0, rows1, rows2, rows3)
    sems = (sem0, sem1, sem2, sem3)

    def drain(rbuf, sem):
        pltpu.make_async_copy(z_hbm.at[stab.at[0]], rbuf, sem).wait()

    for p in range(2):
        def fill_zero(i, carry):
            for k in range(HALF // 16):
                wbuf[i, pl.ds(k * 16, 16)] = jnp.zeros((16,), _f32)
            return carry
        lax.fori_loop(0, CHUNK, fill_zero, 0)
        z0 = sid * ZCHUNK
        for off in (0, 80, 160, 240):
            pltpu.sync_copy(wbuf, acc.at[pl.ds(z0 + off, CHUNK)])
        pltpu.sync_copy(wbuf.at[pl.ds(0, ZCHUNK - 320)],
                        acc.at[pl.ds(z0 + 320, ZCHUNK - 320)])
        plsc.subcore_barrier()

        def sbody(sj, carry):
            base = tile_base + sj * (SUP * CHUNK)
            pltpu.sync_copy(src_hbm.at[pl.ds(base, SUP * CHUNK)], sstage)
            pltpu.sync_copy(dst_hbm.at[pl.ds(base, SUP * CHUNK)], dstage)
            for t in range(SUP):
                for i in range(CHUNK // 16):
                    sl = pl.ds(t * CHUNK + i * 16, 16)
                    stab[t, pl.ds(i * 16, 16)] = sstage[sl] + plane_off
                    d = dstage[sl] - p * PASS_ROWS
                    oob = (d < 0) | (d >= PASS_ROWS)
                    dtab[t, pl.ds(i * 16, 16)] = jnp.where(oob, trash, d)
            for t in range(3):
                pltpu.async_copy(z_hbm.at[stab.at[t]], rows[t], sems[t])
            for t in range(SUP):
                b = t % 4
                drain(rows[b], sems[b])
                pltpu.async_copy(rows[b], acc.at[dtab.at[t]],
                                 sems[b], add=True)
                nt = t + 3
                if nt < SUP:
                    nb = nt % 4
                    if t >= 1:
                        drain(rows[nb], sems[nb])
                    pltpu.async_copy(z_hbm.at[stab.at[nt]], rows[nb],
                                     sems[nb])
            for t in range(SUP - 4, SUP):
                drain(rows[t % 4], sems[t % 4])
            return carry
        lax.fori_loop(0, ept // (SUP * CHUNK), sbody, 0)
        plsc.subcore_barrier()

        row0 = sid * WCHUNK
        out0 = cid * NPAD + p * PASS_ROWS + row0
        for off in (0, 80, 160, 240):
            pltpu.sync_copy(acc.at[pl.ds(row0 + off, CHUNK)], wbuf)
            pltpu.sync_copy(wbuf, out_hbm.at[pl.ds(out0 + off, CHUNK)])
        plsc.subcore_barrier()


@functools.lru_cache(maxsize=1)
def _sc_kernels():
    mesh = plsc.VectorSubcoreMesh(core_axis_name="c", subcore_axis_name="s",
                                  num_cores=NC, num_subcores=NS)
    sc_degree = pl.kernel(
        _sc_degree_body,
        out_type=jax.ShapeDtypeStruct((NC * NPAD, HALF), _f32),
        mesh=mesh,
        scratch_types=[
            pltpu.VMEM((DSUP * CHUNK,), _i32),
            pltpu.VMEM((DSUP, CHUNK), _i32),
            pltpu.VMEM((CHUNK, HALF), _f32),
            pltpu.VMEM((CHUNK, HALF), _f32),
            pltpu.VMEM_SHARED((ACC_ROWS, HALF), _f32),
            pltpu.SemaphoreType.DMA,
        ],
        name="sc_degree",
    )
    sc_aggregate = pl.kernel(
        _sc_aggregate_body,
        out_type=jax.ShapeDtypeStruct((NC * NPAD, HALF), _f32),
        mesh=mesh,
        scratch_types=[
            pltpu.VMEM((SUP * CHUNK,), _i32),
            pltpu.VMEM((SUP * CHUNK,), _i32),
            pltpu.VMEM((SUP, CHUNK), _i32),
            pltpu.VMEM((SUP, CHUNK), _i32),
            pltpu.VMEM((CHUNK, HALF), _f32),
            pltpu.VMEM((CHUNK, HALF), _f32),
            pltpu.VMEM((CHUNK, HALF), _f32),
            pltpu.VMEM((CHUNK, HALF), _f32),
            pltpu.VMEM((CHUNK, HALF), _f32),
            pltpu.VMEM_SHARED((ACC_ROWS, HALF), _f32),
            pltpu.SemaphoreType.DMA,
            pltpu.SemaphoreType.DMA,
            pltpu.SemaphoreType.DMA,
            pltpu.SemaphoreType.DMA,
        ],
        name="sc_aggregate",
    )
    return sc_degree, sc_aggregate


_RB = 1000
_NB = N // _RB


def _tc_a_body(deg_ref, x_ref, w1_ref, z_ref, s_ref):
    d = deg_ref[0] + deg_ref[1] + 1.0
    s = lax.rsqrt(d)
    s_ref[...] = s
    s_col = s[:, :1]
    y = jnp.dot(x_ref[...], w1_ref[...], preferred_element_type=_f32)
    z = s_col * y
    z_ref[0] = z[:, :HALF]
    z_ref[1] = z[:, HALF:]


def _tc_a(deg3, x, W1):
    return pl.pallas_call(
        _tc_a_body,
        grid=(_NB,),
        in_specs=[
            pl.BlockSpec((2, _RB, HALF), lambda i: (0, i, 0)),
            pl.BlockSpec((_RB, DIN), lambda i: (i, 0)),
            pl.BlockSpec((DIN, DH), lambda i: (0, 0)),
        ],
        out_specs=[
            pl.BlockSpec((2, _RB, HALF), lambda i: (0, i, 0)),
            pl.BlockSpec((_RB, HALF), lambda i: (i, 0)),
        ],
        out_shape=[
            jax.ShapeDtypeStruct((2, N, HALF), _f32),
            jax.ShapeDtypeStruct((N, HALF), _f32),
        ],
    )(deg3, x, W1)


def _tc_b_body(agg_ref, z_ref, s_ref, b_ref, w_ref, zout_ref):
    a = jnp.concatenate([agg_ref[0], agg_ref[1]], axis=1)
    z = jnp.concatenate([z_ref[0], z_ref[1]], axis=1)
    s_col = s_ref[...][:, :1]
    h = jax.nn.relu(s_col * (a + z) + b_ref[...])
    y = jnp.dot(h, w_ref[...], preferred_element_type=_f32)
    zo = s_col * y
    zout_ref[0] = zo[:, :HALF]
    zout_ref[1] = zo[:, HALF:]


def _tc_b(agg, z, s, b, W):
    return pl.pallas_call(
        _tc_b_body,
        grid=(_NB,),
        in_specs=[
            pl.BlockSpec((2, _RB, HALF), lambda i: (0, i, 0)),
            pl.BlockSpec((2, _RB, HALF), lambda i: (0, i, 0)),
            pl.BlockSpec((_RB, HALF), lambda i: (i, 0)),
            pl.BlockSpec((1, DH), lambda i: (0, 0)),
            pl.BlockSpec((DH, DH), lambda i: (0, 0)),
        ],
        out_specs=pl.BlockSpec((2, _RB, HALF), lambda i: (0, i, 0)),
        out_shape=jax.ShapeDtypeStruct((2, N, HALF), _f32),
    )(agg, z, s, b, W)


def _tc_c_body(agg_ref, z_ref, s_ref, b_ref, batch_ref, wout_ref, bout_ref,
               out_ref, pooled, cnt):
    i = pl.program_id(0)

    @pl.when(i == 0)
    def _init():
        pooled[...] = jnp.zeros((G, DH), _f32)
        cnt[...] = jnp.zeros((G, 128), _f32)

    a = jnp.concatenate([agg_ref[0], agg_ref[1]], axis=1)
    z = jnp.concatenate([z_ref[0], z_ref[1]], axis=1)
    s_col = s_ref[...][:, :1]
    h = jax.nn.relu(s_col * (a + z) + b_ref[...])

    bblk = batch_ref[0]
    gid = lax.broadcasted_iota(_i32, (G, _RB), 0)
    pt = (gid == bblk).astype(_f32)
    pooled[...] += jnp.dot(pt, h, preferred_element_type=_f32)
    rowsum = jnp.sum(pt, axis=1, keepdims=True)
    cnt[...] += jnp.broadcast_to(rowsum, (G, 128))

    @pl.when(i == _NB - 1)
    def _finish():
        denom = jnp.clip(cnt[...][:, :1], 1.0, None)
        avg = pooled[...] / denom
        logits = jnp.dot(avg, wout_ref[...],
                         preferred_element_type=_f32) + bout_ref[...]
        out_ref[...] = jax.nn.sigmoid(logits)


def _tc_c(agg, z, s, b, batch3, Wout, bout):
    return pl.pallas_call(
        _tc_c_body,
        grid=(_NB,),
        in_specs=[
            pl.BlockSpec((2, _RB, HALF), lambda i: (0, i, 0)),
            pl.BlockSpec((2, _RB, HALF), lambda i: (0, i, 0)),
            pl.BlockSpec((_RB, HALF), lambda i: (i, 0)),
            pl.BlockSpec((1, DH), lambda i: (0, 0)),
            pl.BlockSpec((1, 1, _RB), lambda i: (i, 0, 0)),
            pl.BlockSpec((DH, DOUT), lambda i: (0, 0)),
            pl.BlockSpec((1, DOUT), lambda i: (0, 0)),
        ],
        out_specs=pl.BlockSpec((G, DOUT), lambda i: (0, 0)),
        out_shape=jax.ShapeDtypeStruct((G, DOUT), _f32),
        scratch_shapes=[
            pltpu.VMEM((G, DH), _f32),
            pltpu.VMEM((G, 128), _f32),
        ],
    )(agg, z, s, b, batch3, Wout, bout)


def kernel(x, edge_index, batch, W1, b1, W2, b2, Wout, bout):
    src = edge_index[0].astype(_i32)
    dst = edge_index[1].astype(_i32)
    batch3 = batch.astype(_i32).reshape(_NB, 1, _RB)

    sc_degree, sc_aggregate = _sc_kernels()
    deg2 = sc_degree(dst)
    deg3 = deg2.reshape(2, NPAD, HALF)

    z1, s = _tc_a(deg3, x.astype(_f32), W1)
    agg1 = sc_aggregate(src, dst, z1.reshape(NC * N, HALF))
    z2 = _tc_b(agg1.reshape(2, NPAD, HALF), z1, s, b1.reshape(1, DH), W2)
    agg2 = sc_aggregate(src, dst, z2.reshape(NC * N, HALF))
    return _tc_c(agg2.reshape(2, NPAD, HALF), z2, s, b2.reshape(1, DH),
                 batch3, Wout, bout.reshape(1, DOUT))
